# baseline (speedup 1.0000x reference)
import jax
import jax.numpy as jnp
from jax import lax
from jax.experimental import pallas as pl
from jax.experimental.pallas import tpu as pltpu
from jax.experimental.pallas import tpu_sc as plsc

V = 100000
H = 768
L = 2048
B = 4
N = B * L

NC = 2
NS = 16
NW = NC * NS
ROWS_PER_W = N // NW
CHUNK = 64
NCHUNK = ROWS_PER_W // CHUNK


def _sc_gather_body(table_hbm, idx_hbm, out_hbm, idx_v, rows_v, sems):
    wid = lax.axis_index("s") * NC + lax.axis_index("c")
    base = wid * ROWS_PER_W
    pltpu.sync_copy(idx_hbm.at[pl.ds(base, ROWS_PER_W)], idx_v)

    def gather(c, buf):
        return pltpu.async_copy(
            table_hbm.at[idx_v.at[pl.ds(c * CHUNK, CHUNK)]],
            rows_v.at[buf],
            sems.at[buf],
        )

    cp = gather(0, 0)
    for c in range(NCHUNK):
        nxt = None
        if c + 1 < NCHUNK:
            nxt = gather(c + 1, (c + 1) % 2)
        cp.wait()
        pltpu.sync_copy(rows_v.at[c % 2], out_hbm.at[pl.ds(base + c * CHUNK, CHUNK)])
        cp = nxt


import functools


@functools.cache
def _sc_gather():
    return pl.kernel(
        _sc_gather_body,
        out_type=jax.ShapeDtypeStruct((N, H), jnp.float32),
        mesh=plsc.VectorSubcoreMesh(core_axis_name="c", subcore_axis_name="s"),
        scratch_types=[
            pltpu.VMEM((ROWS_PER_W,), jnp.int32),
            pltpu.VMEM((2, CHUNK, H), jnp.float32),
            pltpu.SemaphoreType.DMA((2,)),
        ],
    )


ROWS_1C = N // NS
NCHUNK_1C = ROWS_1C // CHUNK


def _sc_gather_body_1c(table_hbm, idx_hbm, out_hbm, idx_v, rows_v, sems):
    wid = lax.axis_index("s")
    base = wid * ROWS_1C
    pltpu.sync_copy(idx_hbm.at[pl.ds(base, ROWS_1C)], idx_v)

    def gather(c, buf):
        return pltpu.async_copy(
            table_hbm.at[idx_v.at[pl.ds(c * CHUNK, CHUNK)]],
            rows_v.at[buf],
            sems.at[buf],
        )

    cp = gather(0, 0)
    for c in range(NCHUNK_1C):
        nxt = None
        if c + 1 < NCHUNK_1C:
            nxt = gather(c + 1, (c + 1) % 2)
        cp.wait()
        pltpu.sync_copy(rows_v.at[c % 2], out_hbm.at[pl.ds(base + c * CHUNK, CHUNK)])
        cp = nxt


@functools.cache
def _sc_gather_1c():
    return pl.kernel(
        _sc_gather_body_1c,
        out_type=jax.ShapeDtypeStruct((N, H), jnp.float32),
        mesh=plsc.VectorSubcoreMesh(
            core_axis_name="c", subcore_axis_name="s", num_cores=1
        ),
        scratch_types=[
            pltpu.VMEM((ROWS_1C,), jnp.int32),
            pltpu.VMEM((2, CHUNK, H), jnp.float32),
            pltpu.SemaphoreType.DMA((2,)),
        ],
    )

NSLICE = 4
SLICE = N // NSLICE
SL_ROWS_PER_W = SLICE // NS
SL_NCHUNK = SL_ROWS_PER_W // CHUNK


def _sc_gather_body_sl(table_hbm, idx_hbm, out_hbm, idx_v, rows_v, sems):
    wid = lax.axis_index("s")
    base = wid * SL_ROWS_PER_W
    pltpu.sync_copy(idx_hbm.at[pl.ds(base, SL_ROWS_PER_W)], idx_v)

    def gather(c, buf):
        return pltpu.async_copy(
            table_hbm.at[idx_v.at[pl.ds(c * CHUNK, CHUNK)]],
            rows_v.at[buf],
            sems.at[buf],
        )

    cp = gather(0, 0)
    for c in range(SL_NCHUNK):
        nxt = None
        if c + 1 < SL_NCHUNK:
            nxt = gather(c + 1, (c + 1) % 2)
        cp.wait()
        pltpu.sync_copy(rows_v.at[c % 2], out_hbm.at[pl.ds(base + c * CHUNK, CHUNK)])
        cp = nxt


@functools.cache
def _sc_gather_sl():
    return pl.kernel(
        _sc_gather_body_sl,
        out_type=jax.ShapeDtypeStruct((SLICE, H), jnp.float32),
        mesh=plsc.VectorSubcoreMesh(
            core_axis_name="c", subcore_axis_name="s", num_cores=1
        ),
        scratch_types=[
            pltpu.VMEM((SL_ROWS_PER_W,), jnp.int32),
            pltpu.VMEM((2, CHUNK, H), jnp.float32),
            pltpu.SemaphoreType.DMA((2,)),
        ],
    )


BLK = 256
NBLK = N // BLK
BLK_PER_L = L // BLK


def _tc_ln_body(x_ref, pos_ref, segid_ref, segtab_ref, gb_ref, out_ref):
    x = x_ref[...]
    pos = pos_ref[...]
    seg = segid_ref[0, 0, :]
    s0 = segtab_ref[0, :]
    s1 = segtab_ref[1, :]
    seg_e = jnp.where((seg[:, None] == 0), s0[None, :], s1[None, :])
    x = x + pos + seg_e
    mu = jnp.mean(x, axis=-1, keepdims=True)
    xc = x - mu
    var = jnp.mean(xc * xc, axis=-1, keepdims=True)
    inv = lax.rsqrt(var + 1e-12)
    gamma = gb_ref[0, :]
    beta = gb_ref[1, :]
    out_ref[...] = xc * inv * gamma[None, :] + beta[None, :]


_tc_ln = pl.pallas_call(
    _tc_ln_body,
    grid=(NBLK,),
    in_specs=[
        pl.BlockSpec((BLK, H), lambda i: (i, 0)),
        pl.BlockSpec((BLK, H), lambda i: (i % BLK_PER_L, 0)),
        pl.BlockSpec((1, 1, BLK), lambda i: (i, 0, 0)),
        pl.BlockSpec((2, H), lambda i: (0, 0)),
        pl.BlockSpec((2, H), lambda i: (0, 0)),
    ],
    out_specs=pl.BlockSpec((BLK, H), lambda i: (i, 0)),
    out_shape=jax.ShapeDtypeStruct((N, H), jnp.float32),
)


SL_NBLK = SLICE // BLK

_tc_ln_sl = pl.pallas_call(
    _tc_ln_body,
    grid=(SL_NBLK,),
    in_specs=[
        pl.BlockSpec((BLK, H), lambda i: (i, 0)),
        pl.BlockSpec((BLK, H), lambda i: (i, 0)),
        pl.BlockSpec((1, 1, BLK), lambda i: (i, 0, 0)),
        pl.BlockSpec((2, H), lambda i: (0, 0)),
        pl.BlockSpec((2, H), lambda i: (0, 0)),
    ],
    out_specs=pl.BlockSpec((BLK, H), lambda i: (i, 0)),
    out_shape=jax.ShapeDtypeStruct((SLICE, H), jnp.float32),
)


_BISECT = 3

FC = 16
FNCHUNK = ROWS_PER_W // FC
NJ = H // 16
INV_H = 1.0 / H


def _lane_shuf(v, idx16):
    return lax.gather(
        v,
        idx16.reshape(16, 1),
        lax.GatherDimensionNumbers(
            offset_dims=(), collapsed_slice_dims=(0,), start_index_map=(0,)
        ),
        (1,),
        mode=lax.GatherScatterMode.PROMISE_IN_BOUNDS,
    )


def _lane_bcast(v, r):
    return _lane_shuf(v, jnp.full((16,), r, jnp.int32))


def _rsqrt_newton(v):
    i = lax.bitcast_convert_type(v, jnp.int32)
    y = lax.bitcast_convert_type(
        jnp.int32(0x5F3759DF) - lax.shift_right_logical(i, 1), jnp.float32
    )
    for _ in range(3):
        y = y * (1.5 - 0.5 * v * y * y)
    return y


def _sc_fused_body(
    table_hbm,
    ids_hbm,
    seg_hbm,
    pos_hbm,
    segtab_hbm,
    gamma_hbm,
    beta_hbm,
    out_hbm,
    idx_v,
    seg_v,
    tok_v,
    pos_v,
    obuf_v,
    segtab_v,
    gamma_v,
    beta_v,
    stats_v,
    ab_v,
    g_sems,
    p_sems,
    o_sems,
):
    wid = lax.axis_index("s") * NC + lax.axis_index("c")
    base = wid * ROWS_PER_W
    pos_base = lax.rem(wid, L // ROWS_PER_W) * ROWS_PER_W

    pltpu.sync_copy(ids_hbm.at[pl.ds(base, ROWS_PER_W)], idx_v)
    pltpu.sync_copy(seg_hbm.at[pl.ds(base, ROWS_PER_W)], seg_v)
    pltpu.sync_copy(segtab_hbm, segtab_v)
    pltpu.sync_copy(gamma_hbm, gamma_v)
    pltpu.sync_copy(beta_hbm, beta_v)

    def issue(c, b):
        pltpu.async_copy(
            table_hbm.at[idx_v.at[pl.ds(c * FC, FC)]], tok_v.at[b], g_sems.at[b]
        )
        pltpu.async_copy(
            pos_hbm.at[pl.ds(pos_base + c * FC, FC)], pos_v.at[b], p_sems.at[b]
        )

    issue(0, 0)
    issue(1, 1)

    def chunk_pair(i, carry):
        del carry
        c0 = i * 2
        for db in range(2):
            c = c0 + db
            row0 = c * FC

            @pl.when(c >= 2)
            def _wait_obuf():
                pltpu.make_async_copy(obuf_v.at[db], out_hbm.at[pl.ds(0, FC)],
                                      o_sems.at[db]).wait()

            pltpu.make_async_copy(
                table_hbm.at[idx_v.at[pl.ds(0, FC)]], tok_v.at[db], g_sems.at[db]
            ).wait()
            pltpu.make_async_copy(
                pos_hbm.at[pl.ds(0, FC)], pos_v.at[db], p_sems.at[db]
            ).wait()

            segchunk = seg_v[pl.ds(row0, 16)].astype(jnp.float32)
            iota16 = lax.iota(jnp.int32, 16)
            for hb in range(2):
                HR = FC // 2
                rb = hb * HR
                segf = [_lane_bcast(segchunk, rb + r) for r in range(HR)]

                zeros = jnp.zeros((16,), jnp.float32)
                init = tuple([zeros] * HR) + tuple([zeros] * HR)

                def p1_body(j, acc, rb=rb, db=db, segf=segf):
                    s_acc = list(acc[:HR])
                    q_acc = list(acc[HR:])
                    sl = pl.ds(j * 16, 16)
                    s0j = segtab_v[0, sl]
                    dj = segtab_v[1, sl] - s0j
                    for r in range(HR):
                        x = tok_v[db, rb + r, sl] + pos_v[db, rb + r, sl]
                        x = x + (s0j + segf[r] * dj)
                        tok_v[db, rb + r, sl] = x
                        s_acc[r] = s_acc[r] + x
                        q_acc[r] = q_acc[r] + x * x
                    return tuple(s_acc) + tuple(q_acc)

                acc = plsc.parallel_loop(0, NJ, unroll=2, carry=init)(p1_body)

                abc = []
                bbc = []
                for r in range(HR):
                    ts = acc[r]
                    tq = acc[HR + r]
                    for sh in (1, 2, 4, 8):
                        pidx = jnp.bitwise_xor(iota16, sh)
                        ts = ts + _lane_shuf(ts, pidx)
                        tq = tq + _lane_shuf(tq, pidx)
                    mu = ts * INV_H
                    var = tq * INV_H - mu * mu
                    inv = _rsqrt_newton(var + 1e-12)
                    abc.append(inv)
                    bbc.append(-mu * inv)

                def p2_body(j, rb=rb, db=db, abc=abc, bbc=bbc):
                    sl = pl.ds(j * 16, 16)
                    gj = gamma_v[sl]
                    bj = beta_v[sl]
                    for r in range(HR):
                        x = tok_v[db, rb + r, sl]
                        obuf_v[db, rb + r, sl] = (x * abc[r] + bbc[r]) * gj + bj

                plsc.parallel_loop(0, NJ, unroll=2)(p2_body)

            pltpu.async_copy(
                obuf_v.at[db], out_hbm.at[pl.ds(base + row0, FC)], o_sems.at[db]
            )

            @pl.when(c + 2 < FNCHUNK)
            def _issue_next():
                pltpu.async_copy(
                    table_hbm.at[idx_v.at[pl.ds((c + 2) * FC, FC)]],
                    tok_v.at[db],
                    g_sems.at[db],
                )
                pltpu.async_copy(
                    pos_hbm.at[pl.ds(pos_base + (c + 2) * FC, FC)],
                    pos_v.at[db],
                    p_sems.at[db],
                )

        return 0

    lax.fori_loop(0, FNCHUNK // 2, chunk_pair, 0, unroll=False)

    for db in range(2):
        pltpu.make_async_copy(
            obuf_v.at[db], out_hbm.at[pl.ds(0, FC)], o_sems.at[db]
        ).wait()


@functools.cache
def _sc_fused():
    return pl.kernel(
        _sc_fused_body,
        out_type=jax.ShapeDtypeStruct((N, H), jnp.float32),
        mesh=plsc.VectorSubcoreMesh(core_axis_name="c", subcore_axis_name="s"),
        scratch_types=[
            pltpu.VMEM((ROWS_PER_W,), jnp.int32),
            pltpu.VMEM((ROWS_PER_W,), jnp.int32),
            pltpu.VMEM((2, FC, H), jnp.float32),
            pltpu.VMEM((2, FC, H), jnp.float32),
            pltpu.VMEM((2, FC, H), jnp.float32),
            pltpu.VMEM((2, H), jnp.float32),
            pltpu.VMEM((H,), jnp.float32),
            pltpu.VMEM((H,), jnp.float32),
            pltpu.VMEM((2, 16, 16), jnp.float32),
            pltpu.VMEM((2, 16), jnp.float32),
            pltpu.SemaphoreType.DMA((2,)),
            pltpu.SemaphoreType.DMA((2,)),
            pltpu.SemaphoreType.DMA((2,)),
        ],
    )


@jax.jit
def kernel(input_ids, segment_ids, tok_table, seg_table, pos_table, gamma, beta):
    ids = input_ids.reshape(-1).astype(jnp.int32)
    seg_flat = segment_ids.reshape(-1).astype(jnp.int32)
    out = _sc_fused()(
        tok_table, ids, seg_flat, pos_table, seg_table, gamma, beta
    )
    return out.reshape(B, L, H)


# speedup vs baseline: 1.1096x; 1.1096x over previous
import functools

import jax
import jax.numpy as jnp
from jax import lax
from jax.experimental import pallas as pl
from jax.experimental.pallas import tpu as pltpu
from jax.experimental.pallas import tpu_sc as plsc

V = 100000
H = 768
L = 2048
B = 4
N = B * L

NC = 2
NS = 16
NW = NC * NS
ROWS_PER_W = N // NW

FC = 16
FNCHUNK = ROWS_PER_W // FC
NJ = H // 16
INV_H = 1.0 / H


def _lane_shuf(v, idx16):
    return lax.gather(
        v,
        idx16.reshape(16, 1),
        lax.GatherDimensionNumbers(
            offset_dims=(), collapsed_slice_dims=(0,), start_index_map=(0,)
        ),
        (1,),
        mode=lax.GatherScatterMode.PROMISE_IN_BOUNDS,
    )


def _lane_bcast(v, r):
    return _lane_shuf(v, jnp.full((16,), r, jnp.int32))


def _rsqrt_newton(v):
    i = lax.bitcast_convert_type(v, jnp.int32)
    y = lax.bitcast_convert_type(
        jnp.int32(0x5F3759DF) - lax.shift_right_logical(i, 1), jnp.float32
    )
    for _ in range(3):
        y = y * (1.5 - 0.5 * v * y * y)
    return y


def _sc_fused_body(
    table_hbm,
    ids_hbm,
    seg_hbm,
    pos_hbm,
    segtab_hbm,
    gamma_hbm,
    beta_hbm,
    out_hbm,
    idx_v,
    seg_v,
    tok_v,
    pos_v,
    obuf_v,
    segtab_v,
    gamma_v,
    beta_v,
    g_sems,
    p_sems,
    o_sems,
):
    wid = lax.axis_index("s") * NC + lax.axis_index("c")
    base = wid * ROWS_PER_W
    pos_base = lax.rem(wid, L // ROWS_PER_W) * ROWS_PER_W

    pltpu.sync_copy(ids_hbm.at[pl.ds(base, ROWS_PER_W)], idx_v)
    pltpu.sync_copy(seg_hbm.at[pl.ds(base, ROWS_PER_W)], seg_v)
    pltpu.sync_copy(segtab_hbm, segtab_v)
    pltpu.sync_copy(gamma_hbm, gamma_v)
    pltpu.sync_copy(beta_hbm, beta_v)

    def issue(c, b):
        pltpu.async_copy(
            table_hbm.at[idx_v.at[pl.ds(c * FC, FC)]], tok_v.at[b], g_sems.at[b]
        )
        pltpu.async_copy(
            pos_hbm.at[pl.ds(pos_base + c * FC, FC)], pos_v.at[b], p_sems.at[b]
        )

    issue(0, 0)
    issue(1, 1)

    def chunk_pair(i, carry):
        del carry
        c0 = i * 2
        for db in range(2):
            c = c0 + db
            row0 = c * FC

            @pl.when(c >= 2)
            def _wait_obuf():
                pltpu.make_async_copy(
                    obuf_v.at[db], out_hbm.at[pl.ds(0, FC)], o_sems.at[db]
                ).wait()

            pltpu.make_async_copy(
                table_hbm.at[idx_v.at[pl.ds(0, FC)]], tok_v.at[db], g_sems.at[db]
            ).wait()
            pltpu.make_async_copy(
                pos_hbm.at[pl.ds(0, FC)], pos_v.at[db], p_sems.at[db]
            ).wait()

            segchunk = seg_v[pl.ds(row0, 16)].astype(jnp.float32)
            iota16 = lax.iota(jnp.int32, 16)
            for hb in range(2):
                HR = FC // 2
                rb = hb * HR
                segf = [_lane_bcast(segchunk, rb + r) for r in range(HR)]

                zeros = jnp.zeros((16,), jnp.float32)
                init = tuple([zeros] * HR) + tuple([zeros] * HR)

                def p1_body(j, acc, rb=rb, db=db, segf=segf):
                    s_acc = list(acc[:HR])
                    q_acc = list(acc[HR:])
                    sl = pl.ds(j * 16, 16)
                    s0j = segtab_v[0, sl]
                    dj = segtab_v[1, sl] - s0j
                    for r in range(HR):
                        x = tok_v[db, rb + r, sl] + pos_v[db, rb + r, sl]
                        x = x + (s0j + segf[r] * dj)
                        tok_v[db, rb + r, sl] = x
                        s_acc[r] = s_acc[r] + x
                        q_acc[r] = q_acc[r] + x * x
                    return tuple(s_acc) + tuple(q_acc)

                acc = plsc.parallel_loop(0, NJ, carry=init)(p1_body)

                abc = []
                bbc = []
                for r in range(HR):
                    ts = acc[r]
                    tq = acc[HR + r]
                    for sh in (1, 2, 4, 8):
                        pidx = jnp.bitwise_xor(iota16, sh)
                        ts = ts + _lane_shuf(ts, pidx)
                        tq = tq + _lane_shuf(tq, pidx)
                    mu = ts * INV_H
                    var = tq * INV_H - mu * mu
                    inv = _rsqrt_newton(var + 1e-12)
                    abc.append(inv)
                    bbc.append(-mu * inv)

                def p2_body(j, rb=rb, db=db, abc=abc, bbc=bbc):
                    sl = pl.ds(j * 16, 16)
                    gj = gamma_v[sl]
                    bj = beta_v[sl]
                    for r in range(HR):
                        x = tok_v[db, rb + r, sl]
                        obuf_v[db, rb + r, sl] = (x * abc[r] + bbc[r]) * gj + bj

                plsc.parallel_loop(0, NJ)(p2_body)

            pltpu.async_copy(
                obuf_v.at[db], out_hbm.at[pl.ds(base + row0, FC)], o_sems.at[db]
            )

            @pl.when(c + 2 < FNCHUNK)
            def _issue_next():
                pltpu.async_copy(
                    table_hbm.at[idx_v.at[pl.ds((c + 2) * FC, FC)]],
                    tok_v.at[db],
                    g_sems.at[db],
                )
                pltpu.async_copy(
                    pos_hbm.at[pl.ds(pos_base + (c + 2) * FC, FC)],
                    pos_v.at[db],
                    p_sems.at[db],
                )

        return 0

    lax.fori_loop(0, FNCHUNK // 2, chunk_pair, 0, unroll=False)

    for db in range(2):
        pltpu.make_async_copy(
            obuf_v.at[db], out_hbm.at[pl.ds(0, FC)], o_sems.at[db]
        ).wait()


@functools.cache
def _sc_fused():
    return pl.kernel(
        _sc_fused_body,
        out_type=jax.ShapeDtypeStruct((N, H), jnp.float32),
        mesh=plsc.VectorSubcoreMesh(core_axis_name="c", subcore_axis_name="s"),
        scratch_types=[
            pltpu.VMEM((ROWS_PER_W,), jnp.int32),
            pltpu.VMEM((ROWS_PER_W,), jnp.int32),
            pltpu.VMEM((2, FC, H), jnp.float32),
            pltpu.VMEM((2, FC, H), jnp.float32),
            pltpu.VMEM((2, FC, H), jnp.float32),
            pltpu.VMEM((2, H), jnp.float32),
            pltpu.VMEM((H,), jnp.float32),
            pltpu.VMEM((H,), jnp.float32),
            pltpu.SemaphoreType.DMA((2,)),
            pltpu.SemaphoreType.DMA((2,)),
            pltpu.SemaphoreType.DMA((2,)),
        ],
    )


@jax.jit
def kernel(input_ids, segment_ids, tok_table, seg_table, pos_table, gamma, beta):
    ids = input_ids.reshape(-1).astype(jnp.int32)
    seg_flat = segment_ids.reshape(-1).astype(jnp.int32)
    out = _sc_fused()(tok_table, ids, seg_flat, pos_table, seg_table, gamma, beta)
    return out.reshape(B, L, H)
